# baseline (speedup 1.0000x reference)
import functools
import math

import numpy as np
import jax
import jax.numpy as jnp
from jax import lax
from jax.experimental import pallas as pl
from jax.experimental.pallas import tpu as pltpu
from jax.experimental.pallas import tpu_sc as plsc

VOCAB = 100000
D_MODEL = 128
MAX_LEN = 2048
B = 16
L = 2048
SCALE = math.sqrt(float(D_MODEL))

NC = 2
NS = 16
NW = NC * NS
ROWS_W = L // NW
CHUNKS = B
LANES = 16
NBUF = 8
AHEAD = NBUF - 2


def _positional_encoding_np(length, depth):
    half = depth // 2
    positions = np.arange(length)[:, None].astype(np.float32)
    depths = np.arange(half, dtype=np.float32)[None, :] / float(half)
    angle_rates = 1.0 / np.power(10000.0, depths)
    angle_rads = positions * angle_rates
    return np.concatenate(
        [np.sin(angle_rads), np.cos(angle_rads)], axis=-1
    ).astype(np.float32)


_PE_NP = _positional_encoding_np(MAX_LEN, D_MODEL)


def _body(idx_hbm, pe_hbm, table_hbm, out_hbm, idx_v, pe_sh, buf, sem_in, sem_out, sem_pf):
    c = lax.axis_index("c")
    s = lax.axis_index("s")
    w = s * NC + c

    pltpu.sync_copy(idx_hbm.at[w], idx_v)
    pltpu.sync_copy(pe_hbm.at[pl.ds(w * ROWS_W, ROWS_W)], pe_sh.at[s])

    for k in range(AHEAD):
        pltpu.sync_copy(pe_sh.at[s], buf.at[k])
        pltpu.async_copy(table_hbm.at[idx_v.at[k]], buf.at[k], sem_in, add=True)
    pltpu.async_copy(pe_sh.at[s], buf.at[AHEAD], sem_pf)

    def group(g, carry):
        for j in range(NBUF):
            b = g * NBUF + j
            pltpu.make_async_copy(
                table_hbm.at[idx_v.at[0]], buf.at[j], sem_in
            ).wait()

            @pl.when(b >= 1)
            def _():
                pltpu.make_async_copy(
                    buf.at[j], out_hbm.at[pl.ds(0, ROWS_W)], sem_out
                ).wait()

            @pl.when(b + AHEAD < CHUNKS)
            def _():
                pltpu.make_async_copy(
                    pe_sh.at[s], buf.at[(j + AHEAD) % NBUF], sem_pf
                ).wait()
                pltpu.async_copy(
                    table_hbm.at[idx_v.at[b + AHEAD]],
                    buf.at[(j + AHEAD) % NBUF],
                    sem_in,
                    add=True,
                )

            @pl.when(b + AHEAD + 1 < CHUNKS)
            def _():
                pltpu.async_copy(
                    pe_sh.at[s], buf.at[(j + AHEAD + 1) % NBUF], sem_pf
                )

            def row(r, carry2):
                for cc in range(D_MODEL // LANES):
                    sl = pl.ds(cc * LANES, LANES)
                    buf[j, r, sl] = buf[j, r, sl] * SCALE
                return carry2

            lax.fori_loop(0, ROWS_W, row, 0, unroll=4)
            pltpu.async_copy(
                buf.at[j], out_hbm.at[pl.ds(b * L + w * ROWS_W, ROWS_W)], sem_out
            )
        return carry

    lax.fori_loop(0, CHUNKS // NBUF, group, 0)
    pltpu.make_async_copy(
        buf.at[0], out_hbm.at[pl.ds(0, ROWS_W)], sem_out
    ).wait()


@functools.partial(
    pl.kernel,
    out_type=jax.ShapeDtypeStruct((B * L, D_MODEL), jnp.float32),
    mesh=plsc.VectorSubcoreMesh(core_axis_name="c", subcore_axis_name="s"),
    scratch_types=[
        pltpu.VMEM((CHUNKS, ROWS_W), jnp.int32),
        pltpu.VMEM_SHARED((NS, ROWS_W, D_MODEL), jnp.float32),
        pltpu.VMEM((NBUF, ROWS_W, D_MODEL), jnp.float32),
        pltpu.SemaphoreType.DMA,
        pltpu.SemaphoreType.DMA,
        pltpu.SemaphoreType.DMA,
    ],
)
def _pe_embed(idx_hbm, pe_hbm, table_hbm, out_hbm, idx_v, pe_sh, buf, s_in, s_out, s_pf):
    _body(idx_hbm, pe_hbm, table_hbm, out_hbm, idx_v, pe_sh, buf, s_in, s_out, s_pf)


def kernel(x, table):
    idx = x.astype(jnp.int32)
    idx_arr = idx.reshape(B, NW, ROWS_W).transpose(1, 0, 2)
    out = _pe_embed(idx_arr, jnp.asarray(_PE_NP / SCALE), table)
    return out.reshape(B, L, D_MODEL)


# speedup vs baseline: 1.0270x; 1.0270x over previous
import functools
import math

import numpy as np
import jax
import jax.numpy as jnp
from jax import lax
from jax.experimental import pallas as pl
from jax.experimental.pallas import tpu as pltpu
from jax.experimental.pallas import tpu_sc as plsc

VOCAB = 100000
D_MODEL = 128
MAX_LEN = 2048
B = 16
L = 2048
SCALE = math.sqrt(float(D_MODEL))

NC = 2
NS = 16
NW = NC * NS
ROWS_W = L // NW
CHUNKS = B
LANES = 16
NBUF = 8
AHEAD = NBUF - 2


def _positional_encoding_np(length, depth):
    half = depth // 2
    positions = np.arange(length)[:, None].astype(np.float32)
    depths = np.arange(half, dtype=np.float32)[None, :] / float(half)
    angle_rates = 1.0 / np.power(10000.0, depths)
    angle_rads = positions * angle_rates
    return np.concatenate(
        [np.sin(angle_rads), np.cos(angle_rads)], axis=-1
    ).astype(np.float32)


_PE_NP = _positional_encoding_np(MAX_LEN, D_MODEL)


def _body(idx_hbm, pe_hbm, table_hbm, out_hbm, idx_v, pe_sh, buf, sem_in, sem_out):
    c = lax.axis_index("c")
    s = lax.axis_index("s")
    w = s * NC + c

    pltpu.sync_copy(idx_hbm.at[w], idx_v)
    pltpu.sync_copy(pe_hbm.at[pl.ds(w * ROWS_W, ROWS_W)], pe_sh.at[s])

    for k in range(AHEAD):
        pltpu.sync_copy(pe_sh.at[s], buf.at[k])
        pltpu.async_copy(table_hbm.at[idx_v.at[k]], buf.at[k], sem_in, add=True)

    def group(g, carry):
        for j in range(NBUF):
            b = g * NBUF + j
            pltpu.make_async_copy(
                table_hbm.at[idx_v.at[0]], buf.at[j], sem_in
            ).wait()

            @pl.when(b >= NBUF - AHEAD)
            def _():
                pltpu.make_async_copy(
                    buf.at[j], out_hbm.at[pl.ds(0, ROWS_W)], sem_out
                ).wait()

            @pl.when(b + AHEAD < CHUNKS)
            def _():
                pltpu.sync_copy(pe_sh.at[s], buf.at[(j + AHEAD) % NBUF])
                pltpu.async_copy(
                    table_hbm.at[idx_v.at[b + AHEAD]],
                    buf.at[(j + AHEAD) % NBUF],
                    sem_in,
                    add=True,
                )

            def row(r, carry2):
                for cc in range(D_MODEL // LANES):
                    sl = pl.ds(cc * LANES, LANES)
                    buf[j, r, sl] = buf[j, r, sl] * SCALE
                return carry2

            lax.fori_loop(0, ROWS_W, row, 0, unroll=4)
            pltpu.async_copy(
                buf.at[j], out_hbm.at[pl.ds(b * L + w * ROWS_W, ROWS_W)], sem_out
            )
        return carry

    lax.fori_loop(0, CHUNKS // NBUF, group, 0)
    for _ in range(NBUF - AHEAD):
        pltpu.make_async_copy(
            buf.at[0], out_hbm.at[pl.ds(0, ROWS_W)], sem_out
        ).wait()


@functools.partial(
    pl.kernel,
    out_type=jax.ShapeDtypeStruct((B * L, D_MODEL), jnp.float32),
    mesh=plsc.VectorSubcoreMesh(core_axis_name="c", subcore_axis_name="s"),
    scratch_types=[
        pltpu.VMEM((CHUNKS, ROWS_W), jnp.int32),
        pltpu.VMEM_SHARED((NS, ROWS_W, D_MODEL), jnp.float32),
        pltpu.VMEM((NBUF, ROWS_W, D_MODEL), jnp.float32),
        pltpu.SemaphoreType.DMA,
        pltpu.SemaphoreType.DMA,
    ],
)
def _pe_embed(idx_hbm, pe_hbm, table_hbm, out_hbm, idx_v, pe_sh, buf, s_in, s_out):
    _body(idx_hbm, pe_hbm, table_hbm, out_hbm, idx_v, pe_sh, buf, s_in, s_out)


def kernel(x, table):
    idx = x.astype(jnp.int32)
    idx_arr = idx.reshape(B, NW, ROWS_W).transpose(1, 0, 2)
    out = _pe_embed(idx_arr, jnp.asarray(_PE_NP / SCALE), table)
    return out.reshape(B, L, D_MODEL)
